# baseline (speedup 1.0000x reference)
import functools

import jax
import jax.numpy as jnp
from jax import lax
from jax.experimental import pallas as pl
from jax.experimental.pallas import tpu as pltpu
from jax.experimental.pallas import tpu_sc as plsc

NC = 2
NS = 16
NW = NC * NS
L = 16
WIN = 512


def _rsqrt(x):
    i = plsc.bitcast(x, jnp.int32)
    i = jnp.int32(0x5F3759DF) - lax.shift_right_arithmetic(i, 1)
    y = plsc.bitcast(i, jnp.float32)
    for _ in range(3):
        y = y * (jnp.float32(1.5) - jnp.float32(0.5) * x * y * y)
    return y


def _sqrt(x):
    return x * _rsqrt(x)


def _log(z):
    zi = plsc.bitcast(z, jnp.int32)
    ex = lax.shift_right_arithmetic(zi, 23) - jnp.int32(127)
    mi = (zi & jnp.int32(0x007FFFFF)) | jnp.int32(0x3F800000)
    m = plsc.bitcast(mi, jnp.float32)
    big = m > jnp.float32(1.4142135)
    m = jnp.where(big, m * jnp.float32(0.5), m)
    ex = ex + jnp.where(big, jnp.int32(1), jnp.int32(0))
    s = (m - jnp.float32(1.0)) / (m + jnp.float32(1.0))
    s2 = s * s
    p = s2 * jnp.float32(1.0 / 9.0)
    for c in (1.0 / 7.0, 1.0 / 5.0, 1.0 / 3.0, 1.0):
        p = s2 * p + jnp.float32(c)
    p = jnp.float32(2.0) * s * p
    return ex.astype(jnp.float32) * jnp.float32(0.6931471805599453) + p


def _make_transpose_kernel(N, D):
    assert D == 64
    nfull = (N - D) // WIN
    assert nfull * WIN + D == N
    base_k = nfull // NW
    rem = nfull - base_k * NW
    mesh = plsc.VectorSubcoreMesh(core_axis_name="c", subcore_axis_name="s",
                                  num_cores=NC, num_subcores=NS)

    @functools.partial(
        pl.kernel,
        mesh=mesh,
        out_type=jax.ShapeDtypeStruct((N * D,), jnp.float32),
        compiler_params=pltpu.CompilerParams(needs_layout_passes=False,
                                             use_tc_tiling_on_sc=True),
        scratch_types=[
            pltpu.VMEM((2, D, WIN), jnp.float32),
            pltpu.VMEM((WIN * D,), jnp.float32),
            pltpu.SemaphoreType.DMA,
            pltpu.SemaphoreType.DMA,
        ],
    )
    def tr_kernel(euclT_hbm, tail_hbm, out_hbm, win_v, tout_v, semA, semB):
        wid = lax.axis_index("s") * NC + lax.axis_index("c")
        nw = base_k + jnp.where(wid < rem, 1, 0)
        iota16 = lax.iota(jnp.int32, L)

        def issue(k, m, sem):
            w = k * NW + wid
            pltpu.async_copy(euclT_hbm.at[:, pl.ds(w * WIN, WIN)],
                             win_v.at[m], sem)

        def transpose_window(k, m):
            src = win_v.at[m]

            def col(c, carry):
                for q in range(D // L):
                    vals = plsc.load_gather(src, [q * L + iota16,
                                                  jnp.full((L,), 0, jnp.int32) + c])
                    tout_v[pl.ds(c * D + q * L, L)] = vals
                return carry

            lax.fori_loop(0, WIN, col, 0)
            w = k * NW + wid
            pltpu.sync_copy(tout_v, out_hbm.at[pl.ds(w * (WIN * D), WIN * D)])

        @pl.when(nw > 0)
        def _prologue():
            issue(0, 0, semA)

        nh = (base_k + 2) // 2

        def step(k2, carry):
            k0 = 2 * k2
            k1 = 2 * k2 + 1

            @pl.when(k1 < nw)
            def _issue1():
                issue(k1, 1, semB)

            @pl.when(k0 < nw)
            def _do0():
                pltpu.make_async_copy(euclT_hbm.at[:, pl.ds(0, WIN)],
                                      win_v.at[0], semA).wait()
                transpose_window(k0, 0)

            @pl.when(k1 + 1 < nw)
            def _issue0():
                issue(k1 + 1, 0, semA)

            @pl.when(k1 < nw)
            def _do1():
                pltpu.make_async_copy(euclT_hbm.at[:, pl.ds(0, WIN)],
                                      win_v.at[1], semB).wait()
                transpose_window(k1, 1)

            return carry

        lax.fori_loop(0, nh, step, 0)

        @pl.when(wid == NW - 1)
        def _tail():
            pltpu.sync_copy(tail_hbm, win_v.at[0].at[:, pl.ds(0, 128)])
            src = win_v.at[0]

            def col(c, carry):
                for q in range(D // L):
                    vals = plsc.load_gather(src, [q * L + iota16,
                                                  jnp.full((L,), 0, jnp.int32) + c])
                    tout_v[pl.ds(c * D + q * L, L)] = vals
                return carry

            lax.fori_loop(0, D, col, 0)
            pltpu.sync_copy(tout_v.at[pl.ds(0, D * D)],
                            out_hbm.at[pl.ds((N - D) * D, D * D)])

    return tr_kernel


def _make_gather_kernel(N, D, B):
    assert D == 64 and B % NW == 0
    bpw = B // NW
    ngrp = bpw // L
    nch = bpw // 128
    dh = D // 2
    mesh = plsc.VectorSubcoreMesh(core_axis_name="c", subcore_axis_name="s",
                                  num_cores=NC, num_subcores=NS)

    @functools.partial(
        pl.kernel,
        mesh=mesh,
        out_type=jax.ShapeDtypeStruct((B,), jnp.float32),
        compiler_params=pltpu.CompilerParams(needs_layout_passes=False,
                                             use_tc_tiling_on_sc=False),
        scratch_types=[
            pltpu.VMEM((dh, L), jnp.float32),
            pltpu.VMEM((dh, L), jnp.float32),
            pltpu.VMEM((bpw,), jnp.int32),
            pltpu.VMEM((bpw,), jnp.int32),
            pltpu.VMEM((bpw, D), jnp.float32),
            pltpu.VMEM((bpw, D), jnp.float32),
            pltpu.VMEM((bpw,), jnp.float32),
            pltpu.VMEM((bpw,), jnp.float32),
            pltpu.VMEM((bpw,), jnp.float32),
            pltpu.VMEM((bpw,), jnp.float32),
            pltpu.SemaphoreType.DMA,
        ],
    )
    def sc_kernel(cb_hbm, sb_hbm, uidx_hbm, vidx_hbm, w_hbm, eucl_hbm,
                  bias_hbm, out_hbm, cb_v, sb_v, uidx_v, vidx_v, rows_u,
                  rows_v, w_v, bu_v, bv_v, out_v, sem):
        wid = lax.axis_index("s") * NC + lax.axis_index("c")
        base = wid * bpw
        pltpu.sync_copy(cb_hbm, cb_v)
        pltpu.sync_copy(sb_hbm, sb_v)
        pltpu.sync_copy(uidx_hbm.at[pl.ds(base, bpw)], uidx_v)
        pltpu.sync_copy(vidx_hbm.at[pl.ds(base, bpw)], vidx_v)
        pltpu.sync_copy(w_hbm.at[pl.ds(base, bpw)], w_v)
        cps = []
        for k in range(nch):
            sl = pl.ds(k * 128, 128)
            cps.append(pltpu.async_copy(
                eucl_hbm.at[uidx_v.at[sl]], rows_u.at[sl], sem))
            cps.append(pltpu.async_copy(
                eucl_hbm.at[vidx_v.at[sl]], rows_v.at[sl], sem))
            cps.append(pltpu.async_copy(
                bias_hbm.at[uidx_v.at[sl]], bu_v.at[sl], sem))
            cps.append(pltpu.async_copy(
                bias_hbm.at[vidx_v.at[sl]], bv_v.at[sl], sem))
        for cp in cps:
            cp.wait()

        def group(g, carry):
            p0 = g * L
            idx_p = p0 + lax.iota(jnp.int32, L)
            nu = jnp.zeros((L,), jnp.float32)
            nv = jnp.zeros((L,), jnp.float32)
            dot = jnp.zeros((L,), jnp.float32)
            for j in range(dh):
                de = jnp.full((L,), 2 * j, jnp.int32)
                do = jnp.full((L,), 2 * j + 1, jnp.int32)
                ue = plsc.load_gather(rows_u, [idx_p, de])
                uo = plsc.load_gather(rows_u, [idx_p, do])
                ve = plsc.load_gather(rows_v, [idx_p, de])
                vo = plsc.load_gather(rows_v, [idx_p, do])
                cj = cb_v[j, :]
                sj = sb_v[j, :]
                nu = nu + (ue * ue + uo * uo)
                nv = nv + (ve * ve + vo * vo)
                dot = dot + cj * (ue * ve + uo * vo) + sj * (uo * ve - ue * vo)
            x0u = _sqrt(jnp.float32(1.0) + nu)
            x0v = _sqrt(jnp.float32(1.0) + nv)
            minner = x0u * x0v - dot
            arg = jnp.maximum(minner, jnp.float32(1.0 + 1e-7))
            e = arg - jnp.float32(1.0)
            t = e + _sqrt(e * (e + jnp.float32(2.0)))
            d = _log(jnp.float32(1.0) + t)
            psl = pl.ds(p0, L)
            wv = w_v[psl]
            out_v[psl] = -wv * d * d + bu_v[psl] + bv_v[psl]
            return carry

        lax.fori_loop(0, ngrp, group, 0)
        pltpu.sync_copy(out_v, out_hbm.at[pl.ds(base, bpw)])

    return sc_kernel


def kernel(u_idx, v_idx, w_uv, theta_src, theta_dst, eucl, bias):
    N, D = eucl.shape
    B = u_idx.shape[0]
    phi = theta_dst - theta_src
    cb = jnp.broadcast_to(jnp.cos(phi)[:, None], (D // 2, L))
    sb = jnp.broadcast_to(jnp.sin(phi)[:, None], (D // 2, L))
    tail = jnp.pad(eucl[N - D:, :].T, ((0, 0), (0, 128 - D)))
    flat = _make_transpose_kernel(N, D)(eucl.T, tail)
    table = flat.reshape(N, D)
    return _make_gather_kernel(N, D, B)(
        cb.astype(jnp.float32), sb.astype(jnp.float32),
        u_idx.astype(jnp.int32), v_idx.astype(jnp.int32),
        w_uv, table, bias)


# speedup vs baseline: 2.4836x; 2.4836x over previous
import functools

import jax
import jax.numpy as jnp
from jax import lax
from jax.experimental import pallas as pl
from jax.experimental.pallas import tpu as pltpu
from jax.experimental.pallas import tpu_sc as plsc

NC = 2
NS = 16
NW = NC * NS
L = 16


def _rsqrt(x):
    i = plsc.bitcast(x, jnp.int32)
    i = jnp.int32(0x5F3759DF) - lax.shift_right_arithmetic(i, 1)
    y = plsc.bitcast(i, jnp.float32)
    for _ in range(3):
        y = y * (jnp.float32(1.5) - jnp.float32(0.5) * x * y * y)
    return y


def _sqrt(x):
    return x * _rsqrt(x)


def _log(z):
    zi = plsc.bitcast(z, jnp.int32)
    ex = lax.shift_right_arithmetic(zi, 23) - jnp.int32(127)
    mi = (zi & jnp.int32(0x007FFFFF)) | jnp.int32(0x3F800000)
    m = plsc.bitcast(mi, jnp.float32)
    big = m > jnp.float32(1.4142135)
    m = jnp.where(big, m * jnp.float32(0.5), m)
    ex = ex + jnp.where(big, jnp.int32(1), jnp.int32(0))
    s = (m - jnp.float32(1.0)) / (m + jnp.float32(1.0))
    s2 = s * s
    p = s2 * jnp.float32(1.0 / 9.0)
    for c in (1.0 / 7.0, 1.0 / 5.0, 1.0 / 3.0, 1.0):
        p = s2 * p + jnp.float32(c)
    p = jnp.float32(2.0) * s * p
    return ex.astype(jnp.float32) * jnp.float32(0.6931471805599453) + p


def _make_main_kernel(N, D, B):
    assert D == 64 and B % NW == 0
    bpw = B // NW
    half = bpw // 2
    ngrp = half // L
    nch = half // 128
    dh = D // 2
    mesh = plsc.VectorSubcoreMesh(core_axis_name="c", subcore_axis_name="s",
                                  num_cores=NC, num_subcores=NS)

    @functools.partial(
        pl.kernel,
        mesh=mesh,
        out_type=jax.ShapeDtypeStruct((B,), jnp.float32),
        compiler_params=pltpu.CompilerParams(needs_layout_passes=False,
                                             use_tc_tiling_on_sc=True),
        scratch_types=[
            pltpu.VMEM((dh, L), jnp.float32),
            pltpu.VMEM((dh, L), jnp.float32),
            pltpu.VMEM((bpw,), jnp.int32),
            pltpu.VMEM((bpw,), jnp.int32),
            pltpu.VMEM((half,), jnp.int32),
            pltpu.VMEM((half,), jnp.int32),
            pltpu.VMEM((half, 2 * D), jnp.float32),
            pltpu.VMEM((half, 2 * D), jnp.float32),
            pltpu.VMEM((bpw,), jnp.float32),
            pltpu.VMEM((bpw,), jnp.float32),
            pltpu.SemaphoreType.DMA,
        ],
    )
    def main_kernel(cb_hbm, sb_hbm, uidx_hbm, vidx_hbm, w_hbm, tab_hbm,
                    out_hbm, cb_v, sb_v, uidx_v, vidx_v, ur_v, vr_v, rows_u,
                    rows_v, w_v, out_v, sem):
        wid = lax.axis_index("s") * NC + lax.axis_index("c")
        base = wid * bpw
        pltpu.sync_copy(cb_hbm, cb_v)
        pltpu.sync_copy(sb_hbm, sb_v)
        pltpu.sync_copy(uidx_hbm.at[pl.ds(base, bpw)], uidx_v)
        pltpu.sync_copy(vidx_hbm.at[pl.ds(base, bpw)], vidx_v)
        pltpu.sync_copy(w_hbm.at[pl.ds(base, bpw)], w_v)
        iota16 = lax.iota(jnp.int32, L)

        for hb in (0, half):
            def rowids(g, carry):
                sl = pl.ds(g * L, L)
                iu = uidx_v[pl.ds(hb + g * L, L)]
                iv = vidx_v[pl.ds(hb + g * L, L)]
                ur_v[sl] = lax.shift_right_logical(iu, 1)
                vr_v[sl] = lax.shift_right_logical(iv, 1)
                return carry

            lax.fori_loop(0, ngrp, rowids, 0)
            cps = []
            for k in range(nch):
                sl = pl.ds(k * 128, 128)
                cps.append(pltpu.async_copy(
                    tab_hbm.at[ur_v.at[sl]], rows_u.at[sl], sem))
                cps.append(pltpu.async_copy(
                    tab_hbm.at[vr_v.at[sl]], rows_v.at[sl], sem))
            for cp in cps:
                cp.wait()

            def group(g, carry):
                p0 = g * L
                idx_p = p0 + iota16
                iu = uidx_v[pl.ds(hb + p0, L)]
                iv = vidx_v[pl.ds(hb + p0, L)]
                cu = (iu & jnp.int32(1)) * jnp.int32(D)
                cv = (iv & jnp.int32(1)) * jnp.int32(D)
                nu = jnp.zeros((L,), jnp.float32)
                nv = jnp.zeros((L,), jnp.float32)
                dot = jnp.zeros((L,), jnp.float32)
                for j in range(dh):
                    ue = plsc.load_gather(rows_u, [idx_p, cu + (2 * j)])
                    uo = plsc.load_gather(rows_u, [idx_p, cu + (2 * j + 1)])
                    ve = plsc.load_gather(rows_v, [idx_p, cv + (2 * j)])
                    vo = plsc.load_gather(rows_v, [idx_p, cv + (2 * j + 1)])
                    cj = cb_v[j, :]
                    sj = sb_v[j, :]
                    nu = nu + (ue * ue + uo * uo)
                    nv = nv + (ve * ve + vo * vo)
                    dot = dot + cj * (ue * ve + uo * vo) + sj * (uo * ve - ue * vo)
                x0u = _sqrt(jnp.float32(1.0) + nu)
                x0v = _sqrt(jnp.float32(1.0) + nv)
                minner = x0u * x0v - dot
                arg = jnp.maximum(minner, jnp.float32(1.0 + 1e-7))
                e = arg - jnp.float32(1.0)
                t = e + _sqrt(e * (e + jnp.float32(2.0)))
                d = _log(jnp.float32(1.0) + t)
                wv = w_v[pl.ds(hb + p0, L)]
                out_v[pl.ds(hb + p0, L)] = -wv * d * d
                return carry

            lax.fori_loop(0, ngrp, group, 0)

        pltpu.sync_copy(out_v, out_hbm.at[pl.ds(base, bpw)])

    return main_kernel


def _make_bias_kernel(N, B):
    bpw = B // NW
    nch = bpw // 128
    mesh = plsc.VectorSubcoreMesh(core_axis_name="c", subcore_axis_name="s",
                                  num_cores=NC, num_subcores=NS)

    @functools.partial(
        pl.kernel,
        mesh=mesh,
        out_type=jax.ShapeDtypeStruct((B,), jnp.float32),
        compiler_params=pltpu.CompilerParams(needs_layout_passes=False,
                                             use_tc_tiling_on_sc=False),
        scratch_types=[
            pltpu.VMEM((bpw,), jnp.int32),
            pltpu.VMEM((bpw,), jnp.int32),
            pltpu.VMEM((bpw,), jnp.float32),
            pltpu.VMEM((bpw,), jnp.float32),
            pltpu.VMEM((bpw,), jnp.float32),
            pltpu.SemaphoreType.DMA,
        ],
    )
    def bias_kernel(uidx_hbm, vidx_hbm, score_hbm, bias_hbm, out_hbm,
                    uidx_v, vidx_v, bu_v, bv_v, s_v, sem):
        wid = lax.axis_index("s") * NC + lax.axis_index("c")
        base = wid * bpw
        pltpu.sync_copy(uidx_hbm.at[pl.ds(base, bpw)], uidx_v)
        pltpu.sync_copy(vidx_hbm.at[pl.ds(base, bpw)], vidx_v)
        pltpu.sync_copy(score_hbm.at[pl.ds(base, bpw)], s_v)
        cps = []
        for k in range(nch):
            sl = pl.ds(k * 128, 128)
            cps.append(pltpu.async_copy(
                bias_hbm.at[uidx_v.at[sl]], bu_v.at[sl], sem))
            cps.append(pltpu.async_copy(
                bias_hbm.at[vidx_v.at[sl]], bv_v.at[sl], sem))
        for cp in cps:
            cp.wait()
        for g in range(bpw // L):
            psl = pl.ds(g * L, L)
            s_v[psl] = s_v[psl] + bu_v[psl] + bv_v[psl]
        pltpu.sync_copy(s_v, out_hbm.at[pl.ds(base, bpw)])

    return bias_kernel


def kernel(u_idx, v_idx, w_uv, theta_src, theta_dst, eucl, bias):
    N, D = eucl.shape
    B = u_idx.shape[0]
    phi = theta_dst - theta_src
    cb = jnp.broadcast_to(jnp.cos(phi)[:, None], (D // 2, L))
    sb = jnp.broadcast_to(jnp.sin(phi)[:, None], (D // 2, L))
    tab = eucl.reshape(N // 2, 2 * D)
    ui = u_idx.astype(jnp.int32)
    vi = v_idx.astype(jnp.int32)
    score = _make_main_kernel(N, D, B)(
        cb.astype(jnp.float32), sb.astype(jnp.float32), ui, vi, w_uv, tab)
    return _make_bias_kernel(N, B)(ui, vi, score, bias)
